# Initial kernel scaffold; baseline (speedup 1.0000x reference)
import jax
import jax.numpy as jnp
from jax.experimental import pallas as pl


def kernel(l, w, edge_index, Wt_merge, b_merge):
    raise NotImplementedError("write your pallas kernel here")


# speedup vs baseline: 5.9317x; 5.9317x over previous
import functools

import jax
import jax.numpy as jnp
from jax import lax
from jax.experimental import pallas as pl
from jax.experimental.pallas import tpu as pltpu
from jax.experimental.pallas import tpu_sc as plsc

N = 10000
E = 320000
D = 128

NP = 10240
CHUNK = 128
NSUB = 16
NCH = 157
EPT = NCH * CHUNK
EP = EPT * NSUB
ZROWS = NP // NSUB


def _sc_body(l_hbm, w_hbm, src_hbm, dst_hbm,
             lsum_hbm, wsum_hbm, deg_hbm,
             accum, deg_sh, rows_v, zeros1d, src_v, dst_v, ones_v):
  c = lax.axis_index("c")
  s = lax.axis_index("s")

  def _zero_row(i, _):
    for j in range(D // 16):
      rows_v[i, pl.ds(j * 16, 16)] = jnp.zeros((16,), jnp.float32)
    return 0
  lax.fori_loop(0, CHUNK, _zero_row, 0)

  def _zero_1d(i, _):
    zeros1d[pl.ds(i * 16, 16)] = jnp.zeros((16,), jnp.float32)
    return 0
  lax.fori_loop(0, ZROWS // 16, _zero_1d, 0)

  for j in range(CHUNK // 16):
    ones_v[pl.ds(j * 16, 16)] = jnp.ones((16,), jnp.float32)

  for k in range(ZROWS // CHUNK):
    pltpu.sync_copy(rows_v, accum.at[pl.ds(s * ZROWS + k * CHUNK, CHUNK)])

  @pl.when(c == 0)
  def _():
    pltpu.sync_copy(zeros1d, deg_sh.at[pl.ds(s * ZROWS, ZROWS)])

  plsc.subcore_barrier()

  def _edge_chunk(i, _):
    off = pl.multiple_of(s * EPT + i * CHUNK, CHUNK)
    pltpu.sync_copy(src_hbm.at[pl.ds(off, CHUNK)], src_v)
    pltpu.sync_copy(dst_hbm.at[pl.ds(off, CHUNK)], dst_v)

    @pl.when(c == 0)
    def _():
      pltpu.sync_copy(l_hbm.at[src_v], rows_v)
      pltpu.sync_copy(rows_v, accum.at[dst_v], add=True)
      pltpu.sync_copy(ones_v, deg_sh.at[dst_v], add=True)

    @pl.when(c == 1)
    def _():
      pltpu.sync_copy(w_hbm.at[src_v], rows_v)
      pltpu.sync_copy(rows_v, accum.at[dst_v], add=True)

    return 0
  lax.fori_loop(0, NCH, _edge_chunk, 0)

  plsc.subcore_barrier()

  @pl.when(c == 0)
  def _():
    pltpu.sync_copy(accum.at[pl.ds(s * ZROWS, ZROWS)],
                    lsum_hbm.at[pl.ds(s * ZROWS, ZROWS)])

  @pl.when(c == 1)
  def _():
    pltpu.sync_copy(accum.at[pl.ds(s * ZROWS, ZROWS)],
                    wsum_hbm.at[pl.ds(s * ZROWS, ZROWS)])

  @pl.when((c == 0) & (s == 0))
  def _():
    pltpu.sync_copy(deg_sh, deg_hbm)


_sc_aggregate = pl.kernel(
    _sc_body,
    out_type=(
        jax.ShapeDtypeStruct((NP, D), jnp.float32),
        jax.ShapeDtypeStruct((NP, D), jnp.float32),
        jax.ShapeDtypeStruct((NP,), jnp.float32),
    ),
    mesh=plsc.VectorSubcoreMesh(core_axis_name="c", subcore_axis_name="s"),
    scratch_types=[
        pltpu.VMEM_SHARED((NP, D), jnp.float32),
        pltpu.VMEM_SHARED((NP,), jnp.float32),
        pltpu.VMEM((CHUNK, D), jnp.float32),
        pltpu.VMEM((ZROWS,), jnp.float32),
        pltpu.VMEM((CHUNK,), jnp.int32),
        pltpu.VMEM((CHUNK,), jnp.int32),
        pltpu.VMEM((CHUNK,), jnp.float32),
    ],
    name="mgn_sc_aggregate",
)


def _tc_body(lsum_ref, wsum_ref, deg_ref, l_ref, w_ref, w1_ref, w2_ref, b_ref,
             lnew_ref, wnew_ref):
  dg = deg_ref[...]
  inv = 1.0 / jnp.maximum(dg, 1.0)
  lm = lsum_ref[...] * inv
  wm = wsum_ref[...] * inv
  upd = (
      jnp.dot(lm, w1_ref[...], preferred_element_type=jnp.float32,
              precision=lax.Precision.HIGHEST)
      + jnp.dot(wm, w2_ref[...], preferred_element_type=jnp.float32,
                precision=lax.Precision.HIGHEST)
      + b_ref[...]
  )
  msk = dg > 0.0
  lnew_ref[...] = jnp.where(msk, upd, l_ref[...])
  wnew_ref[...] = jnp.where(msk, wm, w_ref[...])


ROWS_BLK = 400

_tc_merge = pl.pallas_call(
    _tc_body,
    grid=(N // ROWS_BLK,),
    in_specs=[
        pl.BlockSpec((ROWS_BLK, D), lambda i: (i, 0)),
        pl.BlockSpec((ROWS_BLK, D), lambda i: (i, 0)),
        pl.BlockSpec((ROWS_BLK, 1), lambda i: (i, 0)),
        pl.BlockSpec((ROWS_BLK, D), lambda i: (i, 0)),
        pl.BlockSpec((ROWS_BLK, D), lambda i: (i, 0)),
        pl.BlockSpec((D, D), lambda i: (0, 0)),
        pl.BlockSpec((D, D), lambda i: (0, 0)),
        pl.BlockSpec((1, D), lambda i: (0, 0)),
    ],
    out_specs=[
        pl.BlockSpec((ROWS_BLK, D), lambda i: (i, 0)),
        pl.BlockSpec((ROWS_BLK, D), lambda i: (i, 0)),
    ],
    out_shape=[
        jax.ShapeDtypeStruct((N, D), jnp.float32),
        jax.ShapeDtypeStruct((N, D), jnp.float32),
    ],
    name="mgn_tc_merge",
)


@jax.jit
def kernel(l, w, edge_index, Wt_merge, b_merge):
  pad = EP - E
  src_p = jnp.concatenate([edge_index[0], jnp.zeros((pad,), jnp.int32)])
  dst_p = jnp.concatenate(
      [edge_index[1], jnp.full((pad,), NP - 1, jnp.int32)])

  l_sum, w_sum, deg = _sc_aggregate(l, w, src_p, dst_p)

  l_new, w_new = _tc_merge(
      l_sum[:N], w_sum[:N], deg[:N].reshape(N, 1), l, w,
      Wt_merge[:D], Wt_merge[D:], b_merge.reshape(1, D))
  return (l_new, w_new)
